# baseline (speedup 1.0000x reference)
import functools

import jax
import jax.numpy as jnp
from jax import lax
from jax.experimental import pallas as pl
from jax.experimental.pallas import tpu as pltpu
from jax.experimental.pallas import tpu_sc as plsc


def kernel(x, E, P):
    B, S = x.shape
    V, D = E.shape
    DP = 2 * D
    NW = 32
    rows_per_w = B // NW
    C0 = 96
    C1 = S - C0
    nlane = 16
    nbuf = 6
    rem = rows_per_w % nbuf
    glast = (rows_per_w - rem) // nbuf - 1
    assert rem == 2 and glast == 20
    x2 = x.astype(jnp.int32) * 2

    BL = 8192

    def _tpad_body(et_ref, out_ref):
        out_ref[:, :D] = et_ref[...].T

    tpad = pl.pallas_call(
        _tpad_body,
        grid=(pl.cdiv(V, BL),),
        in_specs=[pl.BlockSpec((D, BL), lambda i: (0, i))],
        out_specs=pl.BlockSpec((BL, DP), lambda i: (i, 0)),
        out_shape=jax.ShapeDtypeStruct((V, DP), jnp.float32),
    )
    E2 = tpad(E.T).reshape(2 * V, D)

    mesh = plsc.VectorSubcoreMesh(core_axis_name="c", subcore_axis_name="s")

    @functools.partial(
        pl.kernel,
        mesh=mesh,
        out_type=jax.ShapeDtypeStruct((B, S, DP), jnp.float32),
        compiler_params=pltpu.CompilerParams(use_tc_tiling_on_sc=False),
        scratch_types=(
            [pltpu.VMEM((rows_per_w, S), jnp.int32),
             pltpu.VMEM((S, D), jnp.float32)]
            + [pltpu.VMEM((S, D), jnp.float32) for _ in range(nbuf)]
            + [pltpu.SemaphoreType.DMA for _ in range(2 * nbuf)]
        ),
    )
    def emb(x_hbm, p_hbm, e_hbm, out_hbm, idx_v, p_v, *rest):
        bufs = rest[:nbuf]
        gs = rest[nbuf:2 * nbuf]
        ws = rest[2 * nbuf:]
        cid = lax.axis_index("c")
        sid = lax.axis_index("s")
        wid = sid * 2 + cid
        b0 = wid * rows_per_w
        pltpu.sync_copy(x_hbm.at[pl.ds(b0, rows_per_w)], idx_v)
        pltpu.sync_copy(p_hbm, p_v)

        def g_start(j, b):
            pltpu.async_copy(
                e_hbm.at[idx_v.at[j, pl.ds(0, C0)]], bufs[b].at[pl.ds(0, C0)],
                gs[b])
            pltpu.async_copy(
                e_hbm.at[idx_v.at[j, pl.ds(C0, C1)]],
                bufs[b].at[pl.ds(C0, C1)], gs[b])

        def g_wait(j, b):
            pltpu.make_async_copy(
                e_hbm.at[idx_v.at[j, pl.ds(0, C0)]], bufs[b].at[pl.ds(0, C0)],
                gs[b]).wait()
            pltpu.make_async_copy(
                e_hbm.at[idx_v.at[j, pl.ds(C0, C1)]],
                bufs[b].at[pl.ds(C0, C1)], gs[b]).wait()

        def w_start(j, b):
            pltpu.async_copy(bufs[b], out_hbm.at[b0 + j, :, pl.ds(0, D)],
                             ws[b])

        def w_wait(j, b):
            pltpu.make_async_copy(bufs[b], out_hbm.at[b0 + j, :, pl.ds(0, D)],
                                  ws[b]).wait()

        def add_p(b):
            buf = bufs[b]

            @plsc.parallel_loop(0, S, unroll=8)
            def _(r):
                for w in range(D // nlane):
                    sl = pl.ds(w * nlane, nlane)
                    buf[r, sl] += p_v[r, sl]

        for jj in range(nbuf - 1):
            g_start(jj, jj)

        def step(j, b, do_wwait, do_gstart):
            g_wait(j, b)
            add_p(b)
            w_start(j, b)
            bprev = (b - 1) % nbuf
            if do_wwait:
                w_wait(j - 1, bprev)
            if do_gstart:
                g_start(j + nbuf - 1, bprev)

        for b in range(nbuf):
            step(b, b, do_wwait=(b > 0), do_gstart=True)

        def group(g, carry):
            for b in range(nbuf):
                step(g * nbuf + b, b, do_wwait=True, do_gstart=True)
            return carry

        lax.fori_loop(1, glast, group, 0)

        for b in range(nbuf):
            j = glast * nbuf + b
            step(j, b, do_wwait=True,
                 do_gstart=(j + nbuf - 1 <= rows_per_w - 1))
        for b in range(rem):
            j = (glast + 1) * nbuf + b
            step(j, b, do_wwait=True, do_gstart=False)
        last = rows_per_w - 1
        w_wait(last, last % nbuf)

    out = emb(x2, P, E2)
    return out[:, :, :D]


# speedup vs baseline: 1.0041x; 1.0041x over previous
import functools

import jax
import jax.numpy as jnp
from jax import lax
from jax.experimental import pallas as pl
from jax.experimental.pallas import tpu as pltpu
from jax.experimental.pallas import tpu_sc as plsc


def kernel(x, E, P):
    B, S = x.shape
    V, D = E.shape
    DP = 2 * D
    NW = 32
    rows_per_w = B // NW
    C0 = 96
    C1 = S - C0
    nlane = 16
    nbuf = 4
    ngroups = rows_per_w // nbuf
    x2 = x.astype(jnp.int32) * 2

    BL = 8192

    def _tpad_body(et_ref, out_ref):
        out_ref[:, :D] = et_ref[...].T

    tpad = pl.pallas_call(
        _tpad_body,
        grid=(pl.cdiv(V, BL),),
        in_specs=[pl.BlockSpec((D, BL), lambda i: (0, i))],
        out_specs=pl.BlockSpec((BL, DP), lambda i: (i, 0)),
        out_shape=jax.ShapeDtypeStruct((V, DP), jnp.float32),
    )
    E2 = tpad(E.T).reshape(2 * V, D)

    mesh = plsc.VectorSubcoreMesh(core_axis_name="c", subcore_axis_name="s")

    @functools.partial(
        pl.kernel,
        mesh=mesh,
        out_type=jax.ShapeDtypeStruct((B, S, DP), jnp.float32),
        compiler_params=pltpu.CompilerParams(use_tc_tiling_on_sc=False),
        scratch_types=(
            [pltpu.VMEM((rows_per_w, S), jnp.int32),
             pltpu.VMEM((S, D), jnp.float32)]
            + [pltpu.VMEM((S, D), jnp.float32) for _ in range(nbuf)]
            + [pltpu.SemaphoreType.DMA for _ in range(2 * nbuf)]
        ),
    )
    def emb(x_hbm, p_hbm, e_hbm, out_hbm, idx_v, p_v, *rest):
        bufs = rest[:nbuf]
        gs = rest[nbuf:2 * nbuf]
        ws = rest[2 * nbuf:]
        cid = lax.axis_index("c")
        sid = lax.axis_index("s")
        wid = sid * 2 + cid
        b0 = wid * rows_per_w
        pltpu.sync_copy(x_hbm.at[pl.ds(b0, rows_per_w)], idx_v)
        pltpu.sync_copy(p_hbm, p_v)

        def g_start(j, b):
            pltpu.async_copy(
                e_hbm.at[idx_v.at[j, pl.ds(0, C0)]], bufs[b].at[pl.ds(0, C0)],
                gs[b])
            pltpu.async_copy(
                e_hbm.at[idx_v.at[j, pl.ds(C0, C1)]],
                bufs[b].at[pl.ds(C0, C1)], gs[b])

        def g_wait(j, b):
            pltpu.make_async_copy(
                e_hbm.at[idx_v.at[j, pl.ds(0, C0)]], bufs[b].at[pl.ds(0, C0)],
                gs[b]).wait()
            pltpu.make_async_copy(
                e_hbm.at[idx_v.at[j, pl.ds(C0, C1)]],
                bufs[b].at[pl.ds(C0, C1)], gs[b]).wait()

        def w_start(j, b):
            pltpu.async_copy(bufs[b], out_hbm.at[b0 + j, :, pl.ds(0, D)],
                             ws[b])

        def w_wait(j, b):
            pltpu.make_async_copy(bufs[b], out_hbm.at[b0 + j, :, pl.ds(0, D)],
                                  ws[b]).wait()

        def add_p(b):
            buf = bufs[b]

            @plsc.parallel_loop(0, S, unroll=8)
            def _(r):
                for w in range(D // nlane):
                    sl = pl.ds(w * nlane, nlane)
                    buf[r, sl] += p_v[r, sl]

        for jj in range(nbuf - 1):
            g_start(jj, jj)

        def step(j, b, do_wwait, do_gstart):
            g_wait(j, b)
            add_p(b)
            w_start(j, b)
            bprev = (b - 1) % nbuf
            if do_wwait:
                w_wait(j - 1, bprev)
            if do_gstart:
                g_start(j + nbuf - 1, bprev)

        for b in range(nbuf):
            step(b, b, do_wwait=(b > 0), do_gstart=True)

        def group(g, carry):
            for b in range(nbuf):
                step(g * nbuf + b, b, do_wwait=True, do_gstart=True)
            return carry

        lax.fori_loop(1, ngroups - 1, group, 0)

        for b in range(nbuf):
            j = (ngroups - 1) * nbuf + b
            step(j, b, do_wwait=(b == 0), do_gstart=(b == 0))
        for b in range(nbuf):
            w_wait((ngroups - 1) * nbuf + b, b)

    out = emb(x2, P, E2)
    return out[:, :, :D]
